# baseline (speedup 1.0000x reference)
import jax
import jax.numpy as jnp
from jax.experimental import pallas as pl
from jax.experimental.pallas import tpu as pltpu

_N = 10000
_D = 128
_BI = 200
_NI = _N // _BI


def _fused_kernel(x_ref, adj_ref, w_ref, out_ref):
    i = pl.program_id(0)
    agg = jax.lax.dot_general(
        x_ref[...], adj_ref[...],
        (((1,), (1,)), ((), ())),
        preferred_element_type=jnp.float32,
    )
    upd = jnp.dot(agg, w_ref[...], preferred_element_type=jnp.float32)

    @pl.when(i == 0)
    def _():
        out_ref[...] = upd

    @pl.when(i > 0)
    def _():
        out_ref[...] += upd


def kernel(x, adj, weight):
    return pl.pallas_call(
        _fused_kernel,
        grid=(_NI,),
        in_specs=[
            pl.BlockSpec((_D, _N), lambda i: (0, 0)),
            pl.BlockSpec((_BI, _N), lambda i: (i, 0)),
            pl.BlockSpec((_BI, _D), lambda i: (i, 0)),
        ],
        out_specs=pl.BlockSpec((_D, _D), lambda i: (0, 0)),
        out_shape=jax.ShapeDtypeStruct((_D, _D), jnp.float32),
    )(x, adj, weight)
